# baseline (speedup 1.0000x reference)
import jax
import jax.numpy as jnp
from jax import lax
from jax.experimental import pallas as pl
from jax.experimental.pallas import tpu as pltpu
from jax.experimental.pallas import tpu_sc as plsc

_NUM_CORES = 2
_NUM_SUBCORES = 16
_NUM_WORKERS = _NUM_CORES * _NUM_SUBCORES


def _copy_body(table_hbm, out_hbm):
    wid = lax.axis_index("s") * _NUM_CORES + lax.axis_index("c")
    rows = table_hbm.shape[0] // _NUM_WORKERS
    base = wid * rows
    pltpu.sync_copy(
        table_hbm.at[pl.ds(base, rows), :],
        out_hbm.at[0, pl.ds(base, rows), :],
    )


@jax.jit
def kernel(x, pos_embedding):
    seq_len = x.shape[1]
    d_model = pos_embedding.shape[1]
    mesh = plsc.VectorSubcoreMesh(core_axis_name="c", subcore_axis_name="s")
    fn = pl.kernel(
        _copy_body,
        out_type=jax.ShapeDtypeStruct((1, seq_len, d_model), jnp.float32),
        mesh=mesh,
    )
    return fn(pos_embedding[:seq_len])


# speedup vs baseline: 31.4941x; 31.4941x over previous
import jax
import jax.numpy as jnp
from jax import lax
from jax.experimental import pallas as pl
from jax.experimental.pallas import tpu as pltpu
from jax.experimental.pallas import tpu_sc as plsc

_NUM_CORES = 2
_NUM_SUBCORES = 16
_NUM_WORKERS = _NUM_CORES * _NUM_SUBCORES
_CHUNK_ROWS = 16
_NBUF = 3


def _copy_body(table_hbm, out_hbm, bufs, ld_sems, st_sems):
    wid = lax.axis_index("s") * _NUM_CORES + lax.axis_index("c")
    rows = table_hbm.shape[0] // _NUM_WORKERS
    base = wid * rows
    nchunks = rows // _CHUNK_ROWS

    def load(g, b):
        return pltpu.async_copy(
            table_hbm.at[pl.ds(base + g * _CHUNK_ROWS, _CHUNK_ROWS), :],
            bufs.at[b],
            ld_sems.at[b],
        )

    def store(g, b):
        return pltpu.async_copy(
            bufs.at[b],
            out_hbm.at[0, pl.ds(base + g * _CHUNK_ROWS, _CHUNK_ROWS), :],
            st_sems.at[b],
        )

    loads = {}
    stores = {}
    for g in range(min(_NBUF, nchunks)):
        loads[g] = load(g, g)
    for g in range(nchunks):
        b = g % _NBUF
        loads.pop(g).wait()
        stores[g] = store(g, b)
        ng = g + _NBUF
        if ng < nchunks:
            stores.pop(g).wait()
            loads[ng] = load(ng, b)
    for g in sorted(stores):
        stores.pop(g).wait()


@jax.jit
def kernel(x, pos_embedding):
    seq_len = x.shape[1]
    d_model = pos_embedding.shape[1]
    mesh = plsc.VectorSubcoreMesh(core_axis_name="c", subcore_axis_name="s")
    fn = pl.kernel(
        _copy_body,
        out_type=jax.ShapeDtypeStruct((1, seq_len, d_model), jnp.float32),
        mesh=mesh,
        scratch_types=[
            pltpu.VMEM((_NBUF, _CHUNK_ROWS, d_model), jnp.float32),
            pltpu.SemaphoreType.DMA((_NBUF,)),
            pltpu.SemaphoreType.DMA((_NBUF,)),
        ],
    )
    return fn(pos_embedding[:seq_len])


# speedup vs baseline: 32.0052x; 1.0162x over previous
import jax
import jax.numpy as jnp
from jax import lax
from jax.experimental import pallas as pl
from jax.experimental.pallas import tpu as pltpu
from jax.experimental.pallas import tpu_sc as plsc

_NUM_CORES = 2
_NUM_SUBCORES = 16
_NUM_WORKERS = _NUM_CORES * _NUM_SUBCORES
_CHUNK_ROWS = 16
_NBUF = 3


def _copy_body(table_hbm, out_hbm, bufs, ld_sems, st_sems):
    sid = lax.axis_index("s")
    wid = sid * _NUM_CORES + lax.axis_index("c")
    rows = table_hbm.shape[0] // _NUM_WORKERS
    base = wid * rows
    nchunks = rows // _CHUNK_ROWS

    def load(g, b):
        return pltpu.async_copy(
            table_hbm.at[pl.ds(base + g * _CHUNK_ROWS, _CHUNK_ROWS), :],
            bufs.at[sid, b],
            ld_sems.at[b],
        )

    def store(g, b):
        return pltpu.async_copy(
            bufs.at[sid, b],
            out_hbm.at[0, pl.ds(base + g * _CHUNK_ROWS, _CHUNK_ROWS), :],
            st_sems.at[b],
        )

    loads = {}
    stores = {}
    for g in range(min(_NBUF, nchunks)):
        loads[g] = load(g, g)
    for g in range(nchunks):
        b = g % _NBUF
        loads.pop(g).wait()
        stores[g] = store(g, b)
        ng = g + _NBUF
        if ng < nchunks:
            stores.pop(g).wait()
            loads[ng] = load(ng, b)
    for g in sorted(stores):
        stores.pop(g).wait()


@jax.jit
def kernel(x, pos_embedding):
    seq_len = x.shape[1]
    d_model = pos_embedding.shape[1]
    mesh = plsc.VectorSubcoreMesh(core_axis_name="c", subcore_axis_name="s")
    fn = pl.kernel(
        _copy_body,
        out_type=jax.ShapeDtypeStruct((1, seq_len, d_model), jnp.float32),
        mesh=mesh,
        scratch_types=[
            pltpu.VMEM_SHARED(
                (_NUM_SUBCORES, _NBUF, _CHUNK_ROWS, d_model), jnp.float32
            ),
            pltpu.SemaphoreType.DMA((_NBUF,)),
            pltpu.SemaphoreType.DMA((_NBUF,)),
        ],
    )
    return fn(pos_embedding[:seq_len])
